# baseline (speedup 1.0000x reference)
import functools

import jax
import jax.numpy as jnp
from jax import lax
from jax.experimental import pallas as pl
from jax.experimental.pallas import tpu as pltpu
from jax.experimental.pallas import tpu_sc as plsc

N = 50000
E = 800000
NC = 2
NS = 16
NW = NC * NS
BN = 1000
CS = 4096
NCH = 13
NP2 = NCH * CS
AR = 4352
RPT = 272
WB = 256
EPT0 = 19200
EPT1 = 32000
EPT = 25600
EP = NW * EPT
CHI = 1600
CH = 256
PAD = 256
CAP = 32000
ZRW = 136

_mesh = plsc.VectorSubcoreMesh(core_axis_name="c", subcore_axis_name="s")


@functools.partial(
    pl.kernel,
    mesh=_mesh,
    compiler_params=pltpu.CompilerParams(needs_layout_passes=False),
    out_type=[
        jax.ShapeDtypeStruct((NCH * NW * CAP,), jnp.int32),
        jax.ShapeDtypeStruct((NCH * NW * CAP,), jnp.int32),
        jax.ShapeDtypeStruct((NCH * NW * 16,), jnp.int32),
    ],
    scratch_types=[
        pltpu.VMEM((CHI,), jnp.int32),
        pltpu.VMEM((CHI,), jnp.int32),
        pltpu.VMEM((CAP,), jnp.int32),
        pltpu.VMEM((CAP,), jnp.int32),
        pltpu.VMEM((16,), jnp.int32),
    ],
)
def _bin_edges(srcp, dstp, bsrc, bdst, cnts, s_in, d_in, bufs, bufd, rec):
    cid = lax.axis_index("c")
    tid = lax.axis_index("s")
    wid = cid * NS + tid
    iot = lax.iota(jnp.int32, 16)
    estart = jnp.where(cid == 0, tid * EPT0, NS * EPT0 + tid * EPT1)
    nin = jnp.where(cid == 0, EPT0 // CHI, EPT1 // CHI)

    for c in range(NCH):
        lo = c * CS

        def outer(kc, cnt):
            pltpu.sync_copy(srcp.at[pl.ds(estart + kc * CHI, CHI)], s_in)
            pltpu.sync_copy(dstp.at[pl.ds(estart + kc * CHI, CHI)], d_in)

            def inner(g, cnt):
                s16 = s_in[pl.ds(g * 16, 16)]
                d16 = d_in[pl.ds(g * 16, 16)]
                m = (d16 >= lo) & (d16 < lo + CS)
                mi = jnp.where(m, 1, 0).astype(jnp.int32)
                pos = cnt + plsc.cumsum(mi) - mi
                plsc.store_scatter(bufs, [pos], s16, mask=m)
                plsc.store_scatter(bufd, [pos], d16 - lo, mask=m)
                return cnt + plsc.all_reduce_population_count(m)

            return lax.fori_loop(0, CHI // 16, inner, cnt)

        cnt = lax.fori_loop(0, nin, outer,
                            jnp.zeros((16,), jnp.int32))
        padded = (cnt + (PAD - 1)) & jnp.int32(-PAD)

        @pl.loop(0, PAD // 16)
        def _(k):
            pos = cnt + iot + 16 * k
            m2 = pos < padded
            plsc.store_scatter(bufs, [pos], pos & 1023, mask=m2)
            plsc.store_scatter(bufd, [pos], CS + (pos & 255), mask=m2)

        rec[...] = padded
        pltpu.sync_copy(rec, cnts.at[pl.ds((c * NW + wid) * 16, 16)])
        nk = lax.shift_right_logical(jnp.max(padded), 8)

        def wrb(k, carry):
            pltpu.sync_copy(
                bufs.at[pl.ds(k * CH, CH)],
                bsrc.at[pl.ds((c * NW + wid) * CAP + k * CH, CH)])
            pltpu.sync_copy(
                bufd.at[pl.ds(k * CH, CH)],
                bdst.at[pl.ds((c * NW + wid) * CAP + k * CH, CH)])
            return carry

        lax.fori_loop(0, nk, wrb, jnp.int32(0))


def _seg_sum(S):
    @functools.partial(
        pl.kernel,
        mesh=_mesh,
        compiler_params=pltpu.CompilerParams(needs_layout_passes=False),
        out_type=jax.ShapeDtypeStruct((S, NP2, 128), jnp.float32),
        scratch_types=[
            pltpu.VMEM((CH,), jnp.int32),
            pltpu.VMEM((CH,), jnp.int32),
            pltpu.VMEM((CH,), jnp.int32),
            pltpu.VMEM((CH,), jnp.int32),
            pltpu.VMEM((CH, 128), jnp.float32),
            pltpu.VMEM((CH, 128), jnp.float32),
            pltpu.VMEM((ZRW, 128), jnp.float32),
            pltpu.VMEM((16,), jnp.int32),
            pltpu.VMEM_SHARED((AR, 128), jnp.float32),
            pltpu.SemaphoreType.DMA,
            pltpu.SemaphoreType.DMA,
        ],
    )
    def seg(u, bsrc, bdst, cnts, agg, src_v0, dst_v0, src_v1, dst_v1,
            rows_v0, rows_v1, zeros_v, cnt_v, acc, sem0, sem1):
        cid = lax.axis_index("c")
        tid = lax.axis_index("s")

        @pl.loop(0, ZRW)
        def _(i):
            @pl.loop(0, 8)
            def _(l):
                zeros_v.at[i, pl.ds(l * 16, 16)][...] = (
                    jnp.zeros((16,), jnp.float32))

        if True:
            slow_set = (0, 2, 4, 6, 8, 13, 13, 13)
            fast_set = (1, 3, 5, 7, 9, 10, 11, 12)
            for ci in range(8):
                c = jnp.where(cid == 0, slow_set[ci], fast_set[ci])

                @pl.when(c < NCH)
                def _():
                    for slab in range(S):
                        @pl.loop(0, 2)
                        def _(z):
                            pltpu.sync_copy(
                                zeros_v,
                                acc.at[pl.ds(tid * RPT + z * ZRW, ZRW)])

                        plsc.subcore_barrier()

                        for jj in range(2):
                            j = tid + NS * jj
                            sbase = (c * NW + j) * CAP
                            pltpu.sync_copy(
                                cnts.at[pl.ds((c * NW + j) * 16, 16)], cnt_v)
                            nk = lax.shift_right_logical(
                                jnp.max(cnt_v[...]), 8)

                            @pl.when(nk > 0)
                            def _():
                                pltpu.sync_copy(
                                    bsrc.at[pl.ds(sbase, CH)], src_v0)
                                pltpu.sync_copy(
                                    bdst.at[pl.ds(sbase, CH)], dst_v0)
                                pltpu.async_copy(
                                    u.at[slab].at[src_v0], rows_v0, sem0)

                            def body(i, carry):
                                k = 2 * i
                                b1 = sbase + (k + 1) * CH
                                pltpu.sync_copy(bsrc.at[pl.ds(b1, CH)],
                                                src_v1)
                                pltpu.sync_copy(bdst.at[pl.ds(b1, CH)],
                                                dst_v1)
                                pltpu.async_copy(
                                    u.at[slab].at[src_v1], rows_v1, sem1)
                                pltpu.make_async_copy(
                                    u.at[slab].at[src_v0], rows_v0,
                                    sem0).wait()
                                pltpu.sync_copy(
                                    rows_v0, acc.at[dst_v0], add=True)

                                @pl.when(k + 2 < nk)
                                def _():
                                    b2 = sbase + (k + 2) * CH
                                    pltpu.sync_copy(
                                        bsrc.at[pl.ds(b2, CH)], src_v0)
                                    pltpu.sync_copy(
                                        bdst.at[pl.ds(b2, CH)], dst_v0)
                                    pltpu.async_copy(
                                        u.at[slab].at[src_v0], rows_v0, sem0)

                                pltpu.make_async_copy(
                                    u.at[slab].at[src_v1], rows_v1,
                                    sem1).wait()
                                pltpu.sync_copy(
                                    rows_v1, acc.at[dst_v1], add=True)
                                return carry

                            lax.fori_loop(0, lax.shift_right_logical(nk, 1),
                                          body, jnp.int32(0))

                            @pl.when((nk & 1) == 1)
                            def _():
                                pltpu.make_async_copy(
                                    u.at[slab].at[src_v0], rows_v0,
                                    sem0).wait()
                                pltpu.sync_copy(
                                    rows_v0, acc.at[dst_v0], add=True)

                        plsc.subcore_barrier()
                        pltpu.sync_copy(
                            acc.at[pl.ds(tid * WB, WB)],
                            agg.at[slab].at[pl.ds(c * CS + tid * WB, WB)])
                        plsc.subcore_barrier()

    return seg


def _slab_spec(s):
    return pl.BlockSpec((s, BN, 128), lambda i: (0, i, 0))


def _full_spec(*shape):
    nd = len(shape)
    return pl.BlockSpec(shape, lambda i, _n=nd: (0,) * _n)


def _asm(ref, s):
    return jnp.concatenate([ref[w] for w in range(s)], axis=-1)


def _emit(y, ref, s):
    for w in range(s):
        ref[w] = y[:, w * 128:(w + 1) * 128]


def _stage0(in8, W8, b):
    def body(x_ref, w_ref, b_ref, o_ref):
        y = jnp.dot(x_ref[...], w_ref[...],
                    preferred_element_type=jnp.float32) + b_ref[...]
        _emit(y, o_ref, 1)

    return pl.pallas_call(
        body,
        grid=(N // BN,),
        in_specs=[pl.BlockSpec((BN, 8), lambda i: (i, 0)),
                  _full_spec(8, 128), _full_spec(128)],
        out_specs=_slab_spec(1),
        out_shape=jax.ShapeDtypeStruct((1, N, 128), jnp.float32),
    )(in8, W8, b)


def _stage_mid(x, agg, Wa, ba, Wb, bb, Wn, sout, relu_out):
    sin = x.shape[0]

    def body(*refs):
        if Wn is None:
            x_ref, a_ref, wa_ref, ba_ref, wb_ref, bb_ref, o_ref = refs
        else:
            x_ref, a_ref, wa_ref, ba_ref, wb_ref, bb_ref, wn_ref, o_ref = refs
        h = jnp.dot(_asm(x_ref, sin) + _asm(a_ref, sin), wa_ref[...],
                    preferred_element_type=jnp.float32) + ba_ref[...]
        h = jnp.maximum(h, 0.0)
        t = jnp.dot(h, wb_ref[...],
                    preferred_element_type=jnp.float32) + bb_ref[...]
        if relu_out:
            t = jnp.maximum(t, 0.0)
        if Wn is not None:
            t = jnp.dot(t, wn_ref[...], preferred_element_type=jnp.float32)
        _emit(t, o_ref, sout)

    win = sin * 128
    wh = Wa.shape[1]
    wmid = Wb.shape[1]
    in_specs = [_slab_spec(sin), _slab_spec(sin), _full_spec(win, wh),
                _full_spec(wh), _full_spec(wh, wmid), _full_spec(wmid)]
    args = [x, agg, Wa, ba, Wb, bb]
    if Wn is not None:
        in_specs.append(_full_spec(wmid, sout * 128))
        args.append(Wn)
    return pl.pallas_call(
        body,
        grid=(N // BN,),
        in_specs=in_specs,
        out_specs=_slab_spec(sout),
        out_shape=jax.ShapeDtypeStruct((sout, N, 128), jnp.float32),
    )(*args)


def _stage_post(y, agg, bpre, Wb, bb, Wn, sout):
    sin = y.shape[0]

    def body(y_ref, a_ref, bp_ref, wb_ref, bb_ref, wn_ref, o_ref):
        h = jnp.maximum(_asm(y_ref, sin) + _asm(a_ref, sin) + bp_ref[...],
                        0.0)
        t = jnp.dot(h, wb_ref[...],
                    preferred_element_type=jnp.float32) + bb_ref[...]
        t = jnp.maximum(t, 0.0)
        t = jnp.dot(t, wn_ref[...], preferred_element_type=jnp.float32)
        _emit(t, o_ref, sout)

    win = sin * 128
    wmid = Wb.shape[1]
    return pl.pallas_call(
        body,
        grid=(N // BN,),
        in_specs=[_slab_spec(sin), _slab_spec(sin), _full_spec(win),
                  _full_spec(win, wmid), _full_spec(wmid),
                  _full_spec(wmid, sout * 128)],
        out_specs=_slab_spec(sout),
        out_shape=jax.ShapeDtypeStruct((sout, N, 128), jnp.float32),
    )(y, agg, bpre, Wb, bb, Wn)


def _stage_final(y, agg, bpre, wfold, bconst):
    def body(y_ref, a_ref, bp_ref, w_ref, bc_ref, o_ref):
        h = jnp.maximum(_asm(y_ref, 1) + _asm(a_ref, 1) + bp_ref[...], 0.0)
        o_ref[...] = jnp.dot(h, w_ref[...],
                             preferred_element_type=jnp.float32) + bc_ref[...]

    return pl.pallas_call(
        body,
        grid=(N // BN,),
        in_specs=[_slab_spec(1), _slab_spec(1), _full_spec(128),
                  _full_spec(128, 1), _full_spec(1)],
        out_specs=pl.BlockSpec((BN, 1), lambda i: (i, 0)),
        out_shape=jax.ShapeDtypeStruct((N, 1), jnp.float32),
    )(y, agg, bpre, wfold, bconst)


def kernel(vertices, edge_index, faces, total_area, normals, W_fc1, b_fc1,
           W2a, b2a, W2b, b2b, W3a, b3a, W3b, b3b, W5a, b5a, W5b, b5b,
           W6a, b6a, W6b, b6b, W_fc3, b_fc3):
    npad = EP - E
    pad_idx = jnp.arange(npad, dtype=jnp.int32)
    srcp = jnp.concatenate([edge_index[0], pad_idx % N])
    dstp = jnp.concatenate([edge_index[1], N + (pad_idx % 3000)])

    in8 = jnp.concatenate(
        [vertices, total_area[:, None], normals,
         jnp.zeros((N, 1), jnp.float32)], axis=1)
    W8 = jnp.concatenate([W_fc1, jnp.zeros((1, 128), jnp.float32)], axis=0)
    wfold = W6b @ W_fc3
    bconst = b6b @ W_fc3 + b_fc3

    bsrc, bdst, cnts = _bin_edges(srcp, dstp)

    x1 = _stage0(in8, W8, b_fc1)
    agg1 = _seg_sum(1)(x1, bsrc, bdst, cnts)
    x2 = _stage_mid(x1, agg1, W2a, b2a, W2b, b2b, None, 2, True)
    agg2 = _seg_sum(2)(x2, bsrc, bdst, cnts)
    y5 = _stage_mid(x2, agg2, W3a, b3a, W3b, b3b, W5a, 2, True)
    agg3 = _seg_sum(2)(y5, bsrc, bdst, cnts)
    y6 = _stage_post(y5, agg3, b5a, W5b, b5b, W6a, 1)
    agg4 = _seg_sum(1)(y6, bsrc, bdst, cnts)
    out = _stage_final(y6, agg4, b6a, wfold, bconst)
    return out[:, 0]


# speedup vs baseline: 1.0303x; 1.0303x over previous
import functools

import jax
import jax.numpy as jnp
from jax import lax
from jax.experimental import pallas as pl
from jax.experimental.pallas import tpu as pltpu
from jax.experimental.pallas import tpu_sc as plsc

N = 50000
E = 800000
NC = 2
NS = 16
NW = NC * NS
BN = 1000
CS = 4096
NCH = 13
NP2 = NCH * CS
AR = 4352
RPT = 272
WB = 256
EPT0 = 19200
EPT1 = 32000
EPT = 25600
EP = NW * EPT
CHI = 1600
CH = 256
PAD = 256
CAP = 25600
ZRW = 136

_mesh = plsc.VectorSubcoreMesh(core_axis_name="c", subcore_axis_name="s")


@functools.partial(
    pl.kernel,
    mesh=_mesh,
    compiler_params=pltpu.CompilerParams(needs_layout_passes=False),
    out_type=[
        jax.ShapeDtypeStruct((NCH * NW * CAP,), jnp.int32),
        jax.ShapeDtypeStruct((NCH * NW * CAP,), jnp.int32),
        jax.ShapeDtypeStruct((NCH * NW * 16,), jnp.int32),
    ],
    scratch_types=[
        pltpu.VMEM((CHI,), jnp.int32),
        pltpu.VMEM((CHI,), jnp.int32),
        pltpu.VMEM((CAP,), jnp.int32),
        pltpu.VMEM((CAP,), jnp.int32),
        pltpu.VMEM((16,), jnp.int32),
    ],
)
def _bin_edges(srcp, dstp, bsrc, bdst, cnts, s_in, d_in, bufs, bufd, rec):
    cid = lax.axis_index("c")
    tid = lax.axis_index("s")
    wid = cid * NS + tid
    iot = lax.iota(jnp.int32, 16)
    estart = wid * EPT
    nin = EPT // CHI

    for c in range(NCH):
        lo = c * CS

        def outer(kc, cnt):
            pltpu.sync_copy(srcp.at[pl.ds(estart + kc * CHI, CHI)], s_in)
            pltpu.sync_copy(dstp.at[pl.ds(estart + kc * CHI, CHI)], d_in)

            def inner(g, cnt):
                s16 = s_in[pl.ds(g * 16, 16)]
                d16 = d_in[pl.ds(g * 16, 16)]
                m = (d16 >= lo) & (d16 < lo + CS)
                mi = jnp.where(m, 1, 0).astype(jnp.int32)
                pos = cnt + plsc.cumsum(mi) - mi
                plsc.store_scatter(bufs, [pos], s16, mask=m)
                plsc.store_scatter(bufd, [pos], d16 - lo, mask=m)
                return cnt + plsc.all_reduce_population_count(m)

            return lax.fori_loop(0, CHI // 16, inner, cnt)

        cnt = lax.fori_loop(0, nin, outer,
                            jnp.zeros((16,), jnp.int32))
        padded = (cnt + (PAD - 1)) & jnp.int32(-PAD)

        @pl.loop(0, PAD // 16)
        def _(k):
            pos = cnt + iot + 16 * k
            m2 = pos < padded
            plsc.store_scatter(bufs, [pos], pos & 1023, mask=m2)
            plsc.store_scatter(bufd, [pos], CS + (pos & 255), mask=m2)

        rec[...] = padded
        pltpu.sync_copy(rec, cnts.at[pl.ds((c * NW + wid) * 16, 16)])
        nk = lax.shift_right_logical(jnp.max(padded), 8)

        def wrb(k, carry):
            pltpu.sync_copy(
                bufs.at[pl.ds(k * CH, CH)],
                bsrc.at[pl.ds((c * NW + wid) * CAP + k * CH, CH)])
            pltpu.sync_copy(
                bufd.at[pl.ds(k * CH, CH)],
                bdst.at[pl.ds((c * NW + wid) * CAP + k * CH, CH)])
            return carry

        lax.fori_loop(0, nk, wrb, jnp.int32(0))


def _seg_sum(S):
    @functools.partial(
        pl.kernel,
        mesh=_mesh,
        compiler_params=pltpu.CompilerParams(needs_layout_passes=False),
        out_type=jax.ShapeDtypeStruct((S, NP2, 128), jnp.float32),
        scratch_types=[
            pltpu.VMEM((CH,), jnp.int32),
            pltpu.VMEM((CH,), jnp.int32),
            pltpu.VMEM((CH,), jnp.int32),
            pltpu.VMEM((CH,), jnp.int32),
            pltpu.VMEM((CH, 128), jnp.float32),
            pltpu.VMEM((CH, 128), jnp.float32),
            pltpu.VMEM((ZRW, 128), jnp.float32),
            pltpu.VMEM((16,), jnp.int32),
            pltpu.VMEM_SHARED((AR, 128), jnp.float32),
            pltpu.SemaphoreType.DMA,
            pltpu.SemaphoreType.DMA,
        ],
    )
    def seg(u, bsrc, bdst, cnts, agg, src_v0, dst_v0, src_v1, dst_v1,
            rows_v0, rows_v1, zeros_v, cnt_v, acc, sem0, sem1):
        cid = lax.axis_index("c")
        tid = lax.axis_index("s")

        @pl.loop(0, ZRW)
        def _(i):
            @pl.loop(0, 8)
            def _(l):
                zeros_v.at[i, pl.ds(l * 16, 16)][...] = (
                    jnp.zeros((16,), jnp.float32))

        if True:
            slow_set = (0, 2, 4, 6, 8, 13, 13, 13)
            fast_set = (1, 3, 5, 7, 9, 10, 11, 12)
            for ci in range(8):
                c = jnp.where(cid == 0, slow_set[ci], fast_set[ci])

                @pl.when(c < NCH)
                def _():
                    for slab in range(S):
                        @pl.loop(0, 2)
                        def _(z):
                            pltpu.sync_copy(
                                zeros_v,
                                acc.at[pl.ds(tid * RPT + z * ZRW, ZRW)])

                        plsc.subcore_barrier()

                        for jj in range(2):
                            j = tid + NS * jj
                            sbase = (c * NW + j) * CAP
                            pltpu.sync_copy(
                                cnts.at[pl.ds((c * NW + j) * 16, 16)], cnt_v)
                            nk = lax.shift_right_logical(
                                jnp.max(cnt_v[...]), 8)

                            @pl.when(nk > 0)
                            def _():
                                pltpu.sync_copy(
                                    bsrc.at[pl.ds(sbase, CH)], src_v0)
                                pltpu.sync_copy(
                                    bdst.at[pl.ds(sbase, CH)], dst_v0)
                                pltpu.async_copy(
                                    u.at[slab].at[src_v0], rows_v0, sem0)

                            def body(i, carry):
                                k = 2 * i
                                b1 = sbase + (k + 1) * CH
                                pltpu.sync_copy(bsrc.at[pl.ds(b1, CH)],
                                                src_v1)
                                pltpu.sync_copy(bdst.at[pl.ds(b1, CH)],
                                                dst_v1)
                                pltpu.async_copy(
                                    u.at[slab].at[src_v1], rows_v1, sem1)
                                pltpu.make_async_copy(
                                    u.at[slab].at[src_v0], rows_v0,
                                    sem0).wait()
                                pltpu.sync_copy(
                                    rows_v0, acc.at[dst_v0], add=True)

                                @pl.when(k + 2 < nk)
                                def _():
                                    b2 = sbase + (k + 2) * CH
                                    pltpu.sync_copy(
                                        bsrc.at[pl.ds(b2, CH)], src_v0)
                                    pltpu.sync_copy(
                                        bdst.at[pl.ds(b2, CH)], dst_v0)
                                    pltpu.async_copy(
                                        u.at[slab].at[src_v0], rows_v0, sem0)

                                pltpu.make_async_copy(
                                    u.at[slab].at[src_v1], rows_v1,
                                    sem1).wait()
                                pltpu.sync_copy(
                                    rows_v1, acc.at[dst_v1], add=True)
                                return carry

                            lax.fori_loop(0, lax.shift_right_logical(nk, 1),
                                          body, jnp.int32(0))

                            @pl.when((nk & 1) == 1)
                            def _():
                                pltpu.make_async_copy(
                                    u.at[slab].at[src_v0], rows_v0,
                                    sem0).wait()
                                pltpu.sync_copy(
                                    rows_v0, acc.at[dst_v0], add=True)

                        plsc.subcore_barrier()
                        pltpu.sync_copy(
                            acc.at[pl.ds(tid * WB, WB)],
                            agg.at[slab].at[pl.ds(c * CS + tid * WB, WB)])
                        plsc.subcore_barrier()

    return seg


def _slab_spec(s):
    return pl.BlockSpec((s, BN, 128), lambda i: (0, i, 0))


def _full_spec(*shape):
    nd = len(shape)
    return pl.BlockSpec(shape, lambda i, _n=nd: (0,) * _n)


def _asm(ref, s):
    return jnp.concatenate([ref[w] for w in range(s)], axis=-1)


def _emit(y, ref, s):
    for w in range(s):
        ref[w] = y[:, w * 128:(w + 1) * 128]


def _stage0(in8, W8, b):
    def body(x_ref, w_ref, b_ref, o_ref):
        y = jnp.dot(x_ref[...], w_ref[...],
                    preferred_element_type=jnp.float32) + b_ref[...]
        _emit(y, o_ref, 1)

    return pl.pallas_call(
        body,
        grid=(N // BN,),
        in_specs=[pl.BlockSpec((BN, 8), lambda i: (i, 0)),
                  _full_spec(8, 128), _full_spec(128)],
        out_specs=_slab_spec(1),
        out_shape=jax.ShapeDtypeStruct((1, N, 128), jnp.float32),
    )(in8, W8, b)


def _stage_mid(x, agg, Wa, ba, Wb, bb, Wn, sout, relu_out):
    sin = x.shape[0]

    def body(*refs):
        if Wn is None:
            x_ref, a_ref, wa_ref, ba_ref, wb_ref, bb_ref, o_ref = refs
        else:
            x_ref, a_ref, wa_ref, ba_ref, wb_ref, bb_ref, wn_ref, o_ref = refs
        h = jnp.dot(_asm(x_ref, sin) + _asm(a_ref, sin), wa_ref[...],
                    preferred_element_type=jnp.float32) + ba_ref[...]
        h = jnp.maximum(h, 0.0)
        t = jnp.dot(h, wb_ref[...],
                    preferred_element_type=jnp.float32) + bb_ref[...]
        if relu_out:
            t = jnp.maximum(t, 0.0)
        if Wn is not None:
            t = jnp.dot(t, wn_ref[...], preferred_element_type=jnp.float32)
        _emit(t, o_ref, sout)

    win = sin * 128
    wh = Wa.shape[1]
    wmid = Wb.shape[1]
    in_specs = [_slab_spec(sin), _slab_spec(sin), _full_spec(win, wh),
                _full_spec(wh), _full_spec(wh, wmid), _full_spec(wmid)]
    args = [x, agg, Wa, ba, Wb, bb]
    if Wn is not None:
        in_specs.append(_full_spec(wmid, sout * 128))
        args.append(Wn)
    return pl.pallas_call(
        body,
        grid=(N // BN,),
        in_specs=in_specs,
        out_specs=_slab_spec(sout),
        out_shape=jax.ShapeDtypeStruct((sout, N, 128), jnp.float32),
    )(*args)


def _stage_post(y, agg, bpre, Wb, bb, Wn, sout):
    sin = y.shape[0]

    def body(y_ref, a_ref, bp_ref, wb_ref, bb_ref, wn_ref, o_ref):
        h = jnp.maximum(_asm(y_ref, sin) + _asm(a_ref, sin) + bp_ref[...],
                        0.0)
        t = jnp.dot(h, wb_ref[...],
                    preferred_element_type=jnp.float32) + bb_ref[...]
        t = jnp.maximum(t, 0.0)
        t = jnp.dot(t, wn_ref[...], preferred_element_type=jnp.float32)
        _emit(t, o_ref, sout)

    win = sin * 128
    wmid = Wb.shape[1]
    return pl.pallas_call(
        body,
        grid=(N // BN,),
        in_specs=[_slab_spec(sin), _slab_spec(sin), _full_spec(win),
                  _full_spec(win, wmid), _full_spec(wmid),
                  _full_spec(wmid, sout * 128)],
        out_specs=_slab_spec(sout),
        out_shape=jax.ShapeDtypeStruct((sout, N, 128), jnp.float32),
    )(y, agg, bpre, Wb, bb, Wn)


def _stage_final(y, agg, bpre, wfold, bconst):
    def body(y_ref, a_ref, bp_ref, w_ref, bc_ref, o_ref):
        h = jnp.maximum(_asm(y_ref, 1) + _asm(a_ref, 1) + bp_ref[...], 0.0)
        o_ref[...] = jnp.dot(h, w_ref[...],
                             preferred_element_type=jnp.float32) + bc_ref[...]

    return pl.pallas_call(
        body,
        grid=(N // BN,),
        in_specs=[_slab_spec(1), _slab_spec(1), _full_spec(128),
                  _full_spec(128, 1), _full_spec(1)],
        out_specs=pl.BlockSpec((BN, 1), lambda i: (i, 0)),
        out_shape=jax.ShapeDtypeStruct((N, 1), jnp.float32),
    )(y, agg, bpre, wfold, bconst)


def kernel(vertices, edge_index, faces, total_area, normals, W_fc1, b_fc1,
           W2a, b2a, W2b, b2b, W3a, b3a, W3b, b3b, W5a, b5a, W5b, b5b,
           W6a, b6a, W6b, b6b, W_fc3, b_fc3):
    npad = EP - E
    pad_idx = jnp.arange(npad, dtype=jnp.int32)
    srcp = jnp.concatenate([edge_index[0], pad_idx % N])
    dstp = jnp.concatenate([edge_index[1], N + (pad_idx % 3000)])

    in8 = jnp.concatenate(
        [vertices, total_area[:, None], normals,
         jnp.zeros((N, 1), jnp.float32)], axis=1)
    W8 = jnp.concatenate([W_fc1, jnp.zeros((1, 128), jnp.float32)], axis=0)
    wfold = W6b @ W_fc3
    bconst = b6b @ W_fc3 + b_fc3

    bsrc, bdst, cnts = _bin_edges(srcp, dstp)

    x1 = _stage0(in8, W8, b_fc1)
    agg1 = _seg_sum(1)(x1, bsrc, bdst, cnts)
    x2 = _stage_mid(x1, agg1, W2a, b2a, W2b, b2b, None, 2, True)
    agg2 = _seg_sum(2)(x2, bsrc, bdst, cnts)
    y5 = _stage_mid(x2, agg2, W3a, b3a, W3b, b3b, W5a, 2, True)
    agg3 = _seg_sum(2)(y5, bsrc, bdst, cnts)
    y6 = _stage_post(y5, agg3, b5a, W5b, b5b, W6a, 1)
    agg4 = _seg_sum(1)(y6, bsrc, bdst, cnts)
    out = _stage_final(y6, agg4, b6a, wfold, bconst)
    return out[:, 0]


# speedup vs baseline: 1.2079x; 1.1724x over previous
import functools

import jax
import jax.numpy as jnp
from jax import lax
from jax.experimental import pallas as pl
from jax.experimental.pallas import tpu as pltpu
from jax.experimental.pallas import tpu_sc as plsc

N = 50000
E = 800000
NC = 2
NS = 16
NW = NC * NS
BN = 1000
CS = 4096
NCH = 13
NP2 = NCH * CS
AR = 4352
RPT = 272
WB = 256
EPT0 = 19200
EPT1 = 32000
EPT = 25600
EP = NW * EPT
CHI = 1600
CH = 256
PAD = 256
CAP = 25600
ZRW = 136

_mesh = plsc.VectorSubcoreMesh(core_axis_name="c", subcore_axis_name="s")


@functools.partial(
    pl.kernel,
    mesh=_mesh,
    compiler_params=pltpu.CompilerParams(needs_layout_passes=False),
    out_type=[
        jax.ShapeDtypeStruct((NCH * NW * CAP,), jnp.int32),
        jax.ShapeDtypeStruct((NCH * NW * CAP,), jnp.int32),
        jax.ShapeDtypeStruct((NCH * NW * 16,), jnp.int32),
    ],
    scratch_types=[
        pltpu.VMEM((CHI,), jnp.int32),
        pltpu.VMEM((CHI,), jnp.int32),
        pltpu.VMEM((CAP,), jnp.int32),
        pltpu.VMEM((CAP,), jnp.int32),
        pltpu.VMEM((16,), jnp.int32),
    ],
)
def _bin_edges(srcp, dstp, bsrc, bdst, cnts, s_in, d_in, bufs, bufd, rec):
    cid = lax.axis_index("c")
    tid = lax.axis_index("s")
    wid = cid * NS + tid
    iot = lax.iota(jnp.int32, 16)
    estart = wid * EPT
    nin = EPT // CHI

    for c in range(NCH):
        lo = c * CS

        def outer(kc, cnt):
            pltpu.sync_copy(srcp.at[pl.ds(estart + kc * CHI, CHI)], s_in)
            pltpu.sync_copy(dstp.at[pl.ds(estart + kc * CHI, CHI)], d_in)

            def inner(g, cnt):
                s16 = s_in[pl.ds(g * 16, 16)]
                d16 = d_in[pl.ds(g * 16, 16)]
                m = (d16 >= lo) & (d16 < lo + CS)
                mi = jnp.where(m, 1, 0).astype(jnp.int32)
                pos = cnt + plsc.cumsum(mi) - mi
                plsc.store_scatter(bufs, [pos], s16, mask=m)
                plsc.store_scatter(bufd, [pos], d16 - lo, mask=m)
                return cnt + plsc.all_reduce_population_count(m)

            return lax.fori_loop(0, CHI // 16, inner, cnt)

        cnt = lax.fori_loop(0, nin, outer,
                            jnp.zeros((16,), jnp.int32))
        padded = (cnt + (PAD - 1)) & jnp.int32(-PAD)

        @pl.loop(0, PAD // 16)
        def _(k):
            pos = cnt + iot + 16 * k
            m2 = pos < padded
            plsc.store_scatter(bufs, [pos], pos & 1023, mask=m2)
            plsc.store_scatter(bufd, [pos], CS + (pos & 255), mask=m2)

        rec[...] = padded
        pltpu.sync_copy(rec, cnts.at[pl.ds((c * NW + wid) * 16, 16)])
        nk = lax.shift_right_logical(jnp.max(padded), 8)

        def wrb(k, carry):
            pltpu.sync_copy(
                bufs.at[pl.ds(k * CH, CH)],
                bsrc.at[pl.ds((c * NW + wid) * CAP + k * CH, CH)])
            pltpu.sync_copy(
                bufd.at[pl.ds(k * CH, CH)],
                bdst.at[pl.ds((c * NW + wid) * CAP + k * CH, CH)])
            return carry

        lax.fori_loop(0, nk, wrb, jnp.int32(0))


def _seg_sum(S):
    @functools.partial(
        pl.kernel,
        mesh=_mesh,
        compiler_params=pltpu.CompilerParams(needs_layout_passes=False),
        out_type=jax.ShapeDtypeStruct((S, NP2, 128), jnp.float32),
        scratch_types=[
            pltpu.VMEM((CH,), jnp.int32),
            pltpu.VMEM((CH,), jnp.int32),
            pltpu.VMEM((CH,), jnp.int32),
            pltpu.VMEM((CH,), jnp.int32),
            pltpu.VMEM((CH, 128), jnp.float32),
            pltpu.VMEM((CH, 128), jnp.float32),
            pltpu.VMEM((ZRW, 128), jnp.float32),
            pltpu.VMEM((16,), jnp.int32),
            pltpu.VMEM_SHARED((AR, 128), jnp.float32),
            pltpu.SemaphoreType.DMA,
            pltpu.SemaphoreType.DMA,
        ],
    )
    def seg(u, bsrc, bdst, cnts, agg, src_v0, dst_v0, src_v1, dst_v1,
            rows_v0, rows_v1, zeros_v, cnt_v, acc, sem0, sem1):
        cid = lax.axis_index("c")
        tid = lax.axis_index("s")

        @pl.loop(0, ZRW)
        def _(i):
            @pl.loop(0, 8)
            def _(l):
                zeros_v.at[i, pl.ds(l * 16, 16)][...] = (
                    jnp.zeros((16,), jnp.float32))

        if True:
            slow_set = (0, 2, 4, 6, 8, 12, 13, 13)
            fast_set = (1, 3, 5, 7, 9, 10, 11, 13)
            for ci in range(8):
                c = jnp.where(cid == 0, slow_set[ci], fast_set[ci])

                @pl.when(c < NCH)
                def _():
                    for slab in range(S):
                        @pl.loop(0, 2)
                        def _(z):
                            pltpu.sync_copy(
                                zeros_v,
                                acc.at[pl.ds(tid * RPT + z * ZRW, ZRW)])

                        plsc.subcore_barrier()

                        for jj in range(2):
                            j = tid + NS * jj
                            sbase = (c * NW + j) * CAP
                            pltpu.sync_copy(
                                cnts.at[pl.ds((c * NW + j) * 16, 16)], cnt_v)
                            nk = lax.shift_right_logical(
                                jnp.max(cnt_v[...]), 8)

                            @pl.when(nk > 0)
                            def _():
                                pltpu.sync_copy(
                                    bsrc.at[pl.ds(sbase, CH)], src_v0)
                                pltpu.sync_copy(
                                    bdst.at[pl.ds(sbase, CH)], dst_v0)
                                pltpu.async_copy(
                                    u.at[slab].at[src_v0], rows_v0, sem0)

                            def body(i, carry):
                                k = 2 * i
                                b1 = sbase + (k + 1) * CH
                                pltpu.sync_copy(bsrc.at[pl.ds(b1, CH)],
                                                src_v1)
                                pltpu.sync_copy(bdst.at[pl.ds(b1, CH)],
                                                dst_v1)
                                pltpu.async_copy(
                                    u.at[slab].at[src_v1], rows_v1, sem1)
                                pltpu.make_async_copy(
                                    u.at[slab].at[src_v0], rows_v0,
                                    sem0).wait()
                                pltpu.sync_copy(
                                    rows_v0, acc.at[dst_v0], add=True)

                                @pl.when(k + 2 < nk)
                                def _():
                                    b2 = sbase + (k + 2) * CH
                                    pltpu.sync_copy(
                                        bsrc.at[pl.ds(b2, CH)], src_v0)
                                    pltpu.sync_copy(
                                        bdst.at[pl.ds(b2, CH)], dst_v0)
                                    pltpu.async_copy(
                                        u.at[slab].at[src_v0], rows_v0, sem0)

                                pltpu.make_async_copy(
                                    u.at[slab].at[src_v1], rows_v1,
                                    sem1).wait()
                                pltpu.sync_copy(
                                    rows_v1, acc.at[dst_v1], add=True)
                                return carry

                            lax.fori_loop(0, lax.shift_right_logical(nk, 1),
                                          body, jnp.int32(0))

                            @pl.when((nk & 1) == 1)
                            def _():
                                pltpu.make_async_copy(
                                    u.at[slab].at[src_v0], rows_v0,
                                    sem0).wait()
                                pltpu.sync_copy(
                                    rows_v0, acc.at[dst_v0], add=True)

                        plsc.subcore_barrier()
                        pltpu.sync_copy(
                            acc.at[pl.ds(tid * WB, WB)],
                            agg.at[slab].at[pl.ds(c * CS + tid * WB, WB)])
                        plsc.subcore_barrier()

    return seg


def _slab_spec(s):
    return pl.BlockSpec((s, BN, 128), lambda i: (0, i, 0))


def _full_spec(*shape):
    nd = len(shape)
    return pl.BlockSpec(shape, lambda i, _n=nd: (0,) * _n)


def _asm(ref, s):
    return jnp.concatenate([ref[w] for w in range(s)], axis=-1)


def _emit(y, ref, s):
    for w in range(s):
        ref[w] = y[:, w * 128:(w + 1) * 128]


def _stage0(in8, W8, b):
    def body(x_ref, w_ref, b_ref, o_ref):
        y = jnp.dot(x_ref[...], w_ref[...],
                    preferred_element_type=jnp.float32) + b_ref[...]
        _emit(y, o_ref, 1)

    return pl.pallas_call(
        body,
        grid=(N // BN,),
        in_specs=[pl.BlockSpec((BN, 8), lambda i: (i, 0)),
                  _full_spec(8, 128), _full_spec(128)],
        out_specs=_slab_spec(1),
        out_shape=jax.ShapeDtypeStruct((1, N, 128), jnp.float32),
    )(in8, W8, b)


def _stage_mid(x, agg, Wa, ba, Wb, bb, Wn, sout, relu_out):
    sin = x.shape[0]

    def body(*refs):
        if Wn is None:
            x_ref, a_ref, wa_ref, ba_ref, wb_ref, bb_ref, o_ref = refs
        else:
            x_ref, a_ref, wa_ref, ba_ref, wb_ref, bb_ref, wn_ref, o_ref = refs
        h = jnp.dot(_asm(x_ref, sin) + _asm(a_ref, sin), wa_ref[...],
                    preferred_element_type=jnp.float32) + ba_ref[...]
        h = jnp.maximum(h, 0.0)
        t = jnp.dot(h, wb_ref[...],
                    preferred_element_type=jnp.float32) + bb_ref[...]
        if relu_out:
            t = jnp.maximum(t, 0.0)
        if Wn is not None:
            t = jnp.dot(t, wn_ref[...], preferred_element_type=jnp.float32)
        _emit(t, o_ref, sout)

    win = sin * 128
    wh = Wa.shape[1]
    wmid = Wb.shape[1]
    in_specs = [_slab_spec(sin), _slab_spec(sin), _full_spec(win, wh),
                _full_spec(wh), _full_spec(wh, wmid), _full_spec(wmid)]
    args = [x, agg, Wa, ba, Wb, bb]
    if Wn is not None:
        in_specs.append(_full_spec(wmid, sout * 128))
        args.append(Wn)
    return pl.pallas_call(
        body,
        grid=(N // BN,),
        in_specs=in_specs,
        out_specs=_slab_spec(sout),
        out_shape=jax.ShapeDtypeStruct((sout, N, 128), jnp.float32),
    )(*args)


def _stage_post(y, agg, bpre, Wb, bb, Wn, sout):
    sin = y.shape[0]

    def body(y_ref, a_ref, bp_ref, wb_ref, bb_ref, wn_ref, o_ref):
        h = jnp.maximum(_asm(y_ref, sin) + _asm(a_ref, sin) + bp_ref[...],
                        0.0)
        t = jnp.dot(h, wb_ref[...],
                    preferred_element_type=jnp.float32) + bb_ref[...]
        t = jnp.maximum(t, 0.0)
        t = jnp.dot(t, wn_ref[...], preferred_element_type=jnp.float32)
        _emit(t, o_ref, sout)

    win = sin * 128
    wmid = Wb.shape[1]
    return pl.pallas_call(
        body,
        grid=(N // BN,),
        in_specs=[_slab_spec(sin), _slab_spec(sin), _full_spec(win),
                  _full_spec(win, wmid), _full_spec(wmid),
                  _full_spec(wmid, sout * 128)],
        out_specs=_slab_spec(sout),
        out_shape=jax.ShapeDtypeStruct((sout, N, 128), jnp.float32),
    )(y, agg, bpre, Wb, bb, Wn)


def _stage_final(y, agg, bpre, wfold, bconst):
    def body(y_ref, a_ref, bp_ref, w_ref, bc_ref, o_ref):
        h = jnp.maximum(_asm(y_ref, 1) + _asm(a_ref, 1) + bp_ref[...], 0.0)
        o_ref[...] = jnp.dot(h, w_ref[...],
                             preferred_element_type=jnp.float32) + bc_ref[...]

    return pl.pallas_call(
        body,
        grid=(N // BN,),
        in_specs=[_slab_spec(1), _slab_spec(1), _full_spec(128),
                  _full_spec(128, 1), _full_spec(1)],
        out_specs=pl.BlockSpec((BN, 1), lambda i: (i, 0)),
        out_shape=jax.ShapeDtypeStruct((N, 1), jnp.float32),
    )(y, agg, bpre, wfold, bconst)


def kernel(vertices, edge_index, faces, total_area, normals, W_fc1, b_fc1,
           W2a, b2a, W2b, b2b, W3a, b3a, W3b, b3b, W5a, b5a, W5b, b5b,
           W6a, b6a, W6b, b6b, W_fc3, b_fc3):
    npad = EP - E
    pad_idx = jnp.arange(npad, dtype=jnp.int32)
    srcp = jnp.concatenate([edge_index[0], pad_idx % N])
    dstp = jnp.concatenate([edge_index[1], N + (pad_idx % 3000)])

    in8 = jnp.concatenate(
        [vertices, total_area[:, None], normals,
         jnp.zeros((N, 1), jnp.float32)], axis=1)
    W8 = jnp.concatenate([W_fc1, jnp.zeros((1, 128), jnp.float32)], axis=0)
    wfold = W6b @ W_fc3
    bconst = b6b @ W_fc3 + b_fc3

    bsrc, bdst, cnts = _bin_edges(srcp, dstp)

    x1 = _stage0(in8, W8, b_fc1)
    agg1 = _seg_sum(1)(x1, bsrc, bdst, cnts)
    x2 = _stage_mid(x1, agg1, W2a, b2a, W2b, b2b, None, 2, True)
    agg2 = _seg_sum(2)(x2, bsrc, bdst, cnts)
    y5 = _stage_mid(x2, agg2, W3a, b3a, W3b, b3b, W5a, 2, True)
    agg3 = _seg_sum(2)(y5, bsrc, bdst, cnts)
    y6 = _stage_post(y5, agg3, b5a, W5b, b5b, W6a, 1)
    agg4 = _seg_sum(1)(y6, bsrc, bdst, cnts)
    out = _stage_final(y6, agg4, b6a, wfold, bconst)
    return out[:, 0]


# speedup vs baseline: 1.3193x; 1.0922x over previous
import functools

import jax
import jax.numpy as jnp
from jax import lax
from jax.experimental import pallas as pl
from jax.experimental.pallas import tpu as pltpu
from jax.experimental.pallas import tpu_sc as plsc

N = 50000
E = 800000
NC = 2
NS = 16
NW = NC * NS
BN = 1000
CS = 4096
NCH = 13
NP2 = NCH * CS
AR = 4352
RPT = 272
WB = 256
EPT0 = 19200
EPT1 = 32000
EPT = 25600
EP = NW * EPT
CHI = 1600
CH = 256
PAD = 256
CAP = 25600
ZRW = 136

_mesh = plsc.VectorSubcoreMesh(core_axis_name="c", subcore_axis_name="s")


@functools.partial(
    pl.kernel,
    mesh=_mesh,
    compiler_params=pltpu.CompilerParams(needs_layout_passes=False),
    out_type=[
        jax.ShapeDtypeStruct((NCH * NW * 2 * CAP,), jnp.int32),
        jax.ShapeDtypeStruct((NCH * NW * 16,), jnp.int32),
    ],
    scratch_types=[
        pltpu.VMEM((CHI,), jnp.int32),
        pltpu.VMEM((CHI,), jnp.int32),
        pltpu.VMEM((CAP,), jnp.int32),
        pltpu.VMEM((CAP,), jnp.int32),
        pltpu.VMEM((16,), jnp.int32),
    ],
)
def _bin_edges(srcp, dstp, bpair, cnts, s_in, d_in, bufs, bufd, rec):
    cid = lax.axis_index("c")
    tid = lax.axis_index("s")
    wid = cid * NS + tid
    iot = lax.iota(jnp.int32, 16)
    estart = wid * EPT
    nin = EPT // CHI

    for c in range(NCH):
        lo = c * CS

        def outer(kc, cnt):
            pltpu.sync_copy(srcp.at[pl.ds(estart + kc * CHI, CHI)], s_in)
            pltpu.sync_copy(dstp.at[pl.ds(estart + kc * CHI, CHI)], d_in)

            def inner(g, cnt):
                s16 = s_in[pl.ds(g * 16, 16)]
                d16 = d_in[pl.ds(g * 16, 16)]
                m = (d16 >= lo) & (d16 < lo + CS)
                mi = jnp.where(m, 1, 0).astype(jnp.int32)
                pos = cnt + plsc.cumsum(mi) - mi
                plsc.store_scatter(bufs, [pos], s16, mask=m)
                plsc.store_scatter(bufd, [pos], d16 - lo, mask=m)
                return cnt + plsc.all_reduce_population_count(m)

            return lax.fori_loop(0, CHI // 16, inner, cnt)

        cnt = lax.fori_loop(0, nin, outer,
                            jnp.zeros((16,), jnp.int32))
        padded = (cnt + (PAD - 1)) & jnp.int32(-PAD)

        @pl.loop(0, PAD // 16)
        def _(k):
            pos = cnt + iot + 16 * k
            m2 = pos < padded
            plsc.store_scatter(bufs, [pos], pos & 1023, mask=m2)
            plsc.store_scatter(bufd, [pos], CS + (pos & 255), mask=m2)

        rec[...] = padded
        pltpu.sync_copy(rec, cnts.at[pl.ds((c * NW + wid) * 16, 16)])
        nk = lax.shift_right_logical(jnp.max(padded), 8)

        def wrb(k, carry):
            base = (c * NW + wid) * 2 * CAP + k * 2 * CH
            pltpu.sync_copy(bufs.at[pl.ds(k * CH, CH)],
                            bpair.at[pl.ds(base, CH)])
            pltpu.sync_copy(bufd.at[pl.ds(k * CH, CH)],
                            bpair.at[pl.ds(base + CH, CH)])
            return carry

        lax.fori_loop(0, nk, wrb, jnp.int32(0))


def _seg_sum(S):
    @functools.partial(
        pl.kernel,
        mesh=_mesh,
        compiler_params=pltpu.CompilerParams(needs_layout_passes=False),
        out_type=jax.ShapeDtypeStruct((S, NP2, 128), jnp.float32),
        scratch_types=[
            pltpu.VMEM((2 * CH,), jnp.int32),
            pltpu.VMEM((2 * CH,), jnp.int32),
            pltpu.VMEM((CH, 128), jnp.float32),
            pltpu.VMEM((CH, 128), jnp.float32),
            pltpu.VMEM((ZRW, 128), jnp.float32),
            pltpu.VMEM((16,), jnp.int32),
            pltpu.VMEM_SHARED((AR, 128), jnp.float32),
            pltpu.SemaphoreType.DMA,
            pltpu.SemaphoreType.DMA,
        ],
    )
    def seg(u, bpair, cnts, agg, pair_v0, pair_v1,
            rows_v0, rows_v1, zeros_v, cnt_v, acc, sem0, sem1):
        cid = lax.axis_index("c")
        tid = lax.axis_index("s")

        @pl.loop(0, ZRW)
        def _(i):
            @pl.loop(0, 8)
            def _(l):
                zeros_v.at[i, pl.ds(l * 16, 16)][...] = (
                    jnp.zeros((16,), jnp.float32))

        if True:
            slow_set = (0, 2, 4, 6, 8, 12, 13, 13)
            fast_set = (1, 3, 5, 7, 9, 10, 11, 13)
            for ci in range(8):
                c = jnp.where(cid == 0, slow_set[ci], fast_set[ci])

                @pl.when(c < NCH)
                def _():
                    for slab in range(S):
                        @pl.loop(0, 2)
                        def _(z):
                            pltpu.sync_copy(
                                zeros_v,
                                acc.at[pl.ds(tid * RPT + z * ZRW, ZRW)])

                        plsc.subcore_barrier()

                        for jj in range(2):
                            j = tid + NS * jj
                            sbase = (c * NW + j) * 2 * CAP
                            pltpu.sync_copy(
                                cnts.at[pl.ds((c * NW + j) * 16, 16)], cnt_v)
                            nk = lax.shift_right_logical(
                                jnp.max(cnt_v[...]), 8)

                            @pl.when(nk > 0)
                            def _():
                                pltpu.sync_copy(
                                    bpair.at[pl.ds(sbase, 2 * CH)], pair_v0)
                                pltpu.async_copy(
                                    u.at[slab].at[pair_v0.at[pl.ds(0, CH)]],
                                    rows_v0, sem0)

                            def body(i, carry):
                                k = 2 * i
                                b1 = sbase + (k + 1) * 2 * CH
                                pltpu.sync_copy(
                                    bpair.at[pl.ds(b1, 2 * CH)], pair_v1)
                                pltpu.async_copy(
                                    u.at[slab].at[pair_v1.at[pl.ds(0, CH)]],
                                    rows_v1, sem1)
                                pltpu.make_async_copy(
                                    u.at[slab].at[pair_v0.at[pl.ds(0, CH)]],
                                    rows_v0, sem0).wait()
                                pltpu.sync_copy(
                                    rows_v0,
                                    acc.at[pair_v0.at[pl.ds(CH, CH)]],
                                    add=True)

                                @pl.when(k + 2 < nk)
                                def _():
                                    b2 = sbase + (k + 2) * 2 * CH
                                    pltpu.sync_copy(
                                        bpair.at[pl.ds(b2, 2 * CH)], pair_v0)
                                    pltpu.async_copy(
                                        u.at[slab].at[
                                            pair_v0.at[pl.ds(0, CH)]],
                                        rows_v0, sem0)

                                pltpu.make_async_copy(
                                    u.at[slab].at[pair_v1.at[pl.ds(0, CH)]],
                                    rows_v1, sem1).wait()
                                pltpu.sync_copy(
                                    rows_v1,
                                    acc.at[pair_v1.at[pl.ds(CH, CH)]],
                                    add=True)
                                return carry

                            lax.fori_loop(0, lax.shift_right_logical(nk, 1),
                                          body, jnp.int32(0))

                            @pl.when((nk & 1) == 1)
                            def _():
                                pltpu.make_async_copy(
                                    u.at[slab].at[pair_v0.at[pl.ds(0, CH)]],
                                    rows_v0, sem0).wait()
                                pltpu.sync_copy(
                                    rows_v0,
                                    acc.at[pair_v0.at[pl.ds(CH, CH)]],
                                    add=True)

                        plsc.subcore_barrier()
                        pltpu.sync_copy(
                            acc.at[pl.ds(tid * WB, WB)],
                            agg.at[slab].at[pl.ds(c * CS + tid * WB, WB)])
                        plsc.subcore_barrier()

    return seg


def _slab_spec(s):
    return pl.BlockSpec((s, BN, 128), lambda i: (0, i, 0))


def _full_spec(*shape):
    nd = len(shape)
    return pl.BlockSpec(shape, lambda i, _n=nd: (0,) * _n)


def _asm(ref, s):
    return jnp.concatenate([ref[w] for w in range(s)], axis=-1)


def _emit(y, ref, s):
    for w in range(s):
        ref[w] = y[:, w * 128:(w + 1) * 128]


def _stage0(in8, W8, b):
    def body(x_ref, w_ref, b_ref, o_ref):
        y = jnp.dot(x_ref[...], w_ref[...],
                    preferred_element_type=jnp.float32) + b_ref[...]
        _emit(y, o_ref, 1)

    return pl.pallas_call(
        body,
        grid=(N // BN,),
        in_specs=[pl.BlockSpec((BN, 8), lambda i: (i, 0)),
                  _full_spec(8, 128), _full_spec(128)],
        out_specs=_slab_spec(1),
        out_shape=jax.ShapeDtypeStruct((1, N, 128), jnp.float32),
    )(in8, W8, b)


def _stage_mid(x, agg, Wa, ba, Wb, bb, Wn, sout, relu_out):
    sin = x.shape[0]

    def body(*refs):
        if Wn is None:
            x_ref, a_ref, wa_ref, ba_ref, wb_ref, bb_ref, o_ref = refs
        else:
            x_ref, a_ref, wa_ref, ba_ref, wb_ref, bb_ref, wn_ref, o_ref = refs
        h = jnp.dot(_asm(x_ref, sin) + _asm(a_ref, sin), wa_ref[...],
                    preferred_element_type=jnp.float32) + ba_ref[...]
        h = jnp.maximum(h, 0.0)
        t = jnp.dot(h, wb_ref[...],
                    preferred_element_type=jnp.float32) + bb_ref[...]
        if relu_out:
            t = jnp.maximum(t, 0.0)
        if Wn is not None:
            t = jnp.dot(t, wn_ref[...], preferred_element_type=jnp.float32)
        _emit(t, o_ref, sout)

    win = sin * 128
    wh = Wa.shape[1]
    wmid = Wb.shape[1]
    in_specs = [_slab_spec(sin), _slab_spec(sin), _full_spec(win, wh),
                _full_spec(wh), _full_spec(wh, wmid), _full_spec(wmid)]
    args = [x, agg, Wa, ba, Wb, bb]
    if Wn is not None:
        in_specs.append(_full_spec(wmid, sout * 128))
        args.append(Wn)
    return pl.pallas_call(
        body,
        grid=(N // BN,),
        in_specs=in_specs,
        out_specs=_slab_spec(sout),
        out_shape=jax.ShapeDtypeStruct((sout, N, 128), jnp.float32),
    )(*args)


def _stage_post(y, agg, bpre, Wb, bb, Wn, sout):
    sin = y.shape[0]

    def body(y_ref, a_ref, bp_ref, wb_ref, bb_ref, wn_ref, o_ref):
        h = jnp.maximum(_asm(y_ref, sin) + _asm(a_ref, sin) + bp_ref[...],
                        0.0)
        t = jnp.dot(h, wb_ref[...],
                    preferred_element_type=jnp.float32) + bb_ref[...]
        t = jnp.maximum(t, 0.0)
        t = jnp.dot(t, wn_ref[...], preferred_element_type=jnp.float32)
        _emit(t, o_ref, sout)

    win = sin * 128
    wmid = Wb.shape[1]
    return pl.pallas_call(
        body,
        grid=(N // BN,),
        in_specs=[_slab_spec(sin), _slab_spec(sin), _full_spec(win),
                  _full_spec(win, wmid), _full_spec(wmid),
                  _full_spec(wmid, sout * 128)],
        out_specs=_slab_spec(sout),
        out_shape=jax.ShapeDtypeStruct((sout, N, 128), jnp.float32),
    )(y, agg, bpre, Wb, bb, Wn)


def _stage_final(y, agg, bpre, wfold, bconst):
    def body(y_ref, a_ref, bp_ref, w_ref, bc_ref, o_ref):
        h = jnp.maximum(_asm(y_ref, 1) + _asm(a_ref, 1) + bp_ref[...], 0.0)
        o_ref[...] = jnp.dot(h, w_ref[...],
                             preferred_element_type=jnp.float32) + bc_ref[...]

    return pl.pallas_call(
        body,
        grid=(N // BN,),
        in_specs=[_slab_spec(1), _slab_spec(1), _full_spec(128),
                  _full_spec(128, 1), _full_spec(1)],
        out_specs=pl.BlockSpec((BN, 1), lambda i: (i, 0)),
        out_shape=jax.ShapeDtypeStruct((N, 1), jnp.float32),
    )(y, agg, bpre, wfold, bconst)


def kernel(vertices, edge_index, faces, total_area, normals, W_fc1, b_fc1,
           W2a, b2a, W2b, b2b, W3a, b3a, W3b, b3b, W5a, b5a, W5b, b5b,
           W6a, b6a, W6b, b6b, W_fc3, b_fc3):
    npad = EP - E
    pad_idx = jnp.arange(npad, dtype=jnp.int32)
    srcp = jnp.concatenate([edge_index[0], pad_idx % N])
    dstp = jnp.concatenate([edge_index[1], N + (pad_idx % 3000)])

    in8 = jnp.concatenate(
        [vertices, total_area[:, None], normals,
         jnp.zeros((N, 1), jnp.float32)], axis=1)
    W8 = jnp.concatenate([W_fc1, jnp.zeros((1, 128), jnp.float32)], axis=0)
    wfold = W6b @ W_fc3
    bconst = b6b @ W_fc3 + b_fc3

    bpair, cnts = _bin_edges(srcp, dstp)

    x1 = _stage0(in8, W8, b_fc1)
    agg1 = _seg_sum(1)(x1, bpair, cnts)
    x2 = _stage_mid(x1, agg1, W2a, b2a, W2b, b2b, None, 2, True)
    agg2 = _seg_sum(2)(x2, bpair, cnts)
    y5 = _stage_mid(x2, agg2, W3a, b3a, W3b, b3b, W5a, 2, True)
    agg3 = _seg_sum(2)(y5, bpair, cnts)
    y6 = _stage_post(y5, agg3, b5a, W5b, b5b, W6a, 1)
    agg4 = _seg_sum(1)(y6, bpair, cnts)
    out = _stage_final(y6, agg4, b6a, wfold, bconst)
    return out[:, 0]
